# baseline (speedup 1.0000x reference)
import functools

import jax
import jax.numpy as jnp
from jax import lax
from jax.experimental import pallas as pl
from jax.experimental.pallas import tpu as pltpu
from jax.experimental.pallas import tpu_sc as plsc

CH = 128
NBUF = 3


@functools.cache
def _build(N, V, D, n_ch, NC, NS):
  NW = NC * NS
  D2 = 2 * D
  mesh = plsc.VectorSubcoreMesh(core_axis_name="c", subcore_axis_name="s")

  @functools.partial(
      pl.kernel,
      mesh=mesh,
      out_type=jax.ShapeDtypeStruct((N // CH, CH, D), jnp.float32),
      scratch_types=[
          pltpu.VMEM((n_ch, CH), jnp.int32),
          pltpu.VMEM((NBUF, CH, D2), jnp.float32),
          pltpu.VMEM((NBUF, CH, D), jnp.float32),
          [pltpu.SemaphoreType.DMA] * NBUF,
          [pltpu.SemaphoreType.DMA] * NBUF,
      ],
  )
  def k(idx_hbm, table_hbm, out_hbm, idx_v, wide, comp, gsems, wsems):
    wid = lax.axis_index("s") * NC + lax.axis_index("c")
    base = wid * n_ch
    pltpu.sync_copy(idx_hbm.at[wid], idx_v)

    def gather(c, b):
      pltpu.async_copy(table_hbm.at[idx_v.at[c]], wide.at[b], gsems[b])

    def wait_gather(c, b):
      pltpu.make_async_copy(
          table_hbm.at[idx_v.at[c]], wide.at[b], gsems[b]).wait()

    def select(b):
      @pl.loop(0, CH // 16)
      def _(g):
        for rr in range(16):
          r = g * 16 + rr
          for j in range(D // 16):
            comp[b, r, pl.ds(16 * j, 16)] = wide[b, r, pl.ds(16 * j, 16)]

    def write(c, b):
      pltpu.async_copy(comp.at[b], out_hbm.at[base + c], wsems[b])

    def wait_write(c, b):
      pltpu.make_async_copy(
          comp.at[b], out_hbm.at[base + c], wsems[b]).wait()

    gather(0, 0)
    gather(1, 1)

    n_loop = ((n_ch + NBUF - 1) // NBUF) * NBUF

    @pl.loop(0, n_loop, step=NBUF)
    def _(j):
      for b in range(NBUF):
        c = j + b

        @pl.when(jnp.logical_and(c - 2 >= 0, c - 2 < n_ch))
        def _():
          wait_write(c - 2, (b - 2) % NBUF)

        @pl.when(c + 2 < n_ch)
        def _():
          gather(c + 2, (b + 2) % NBUF)

        @pl.when(c < n_ch)
        def _():
          wait_gather(c, b)
          select(b)
          write(c, b)

    for c in range(max(0, n_loop - 2), n_ch):
      wait_write(c, c % NBUF)

  return k


def kernel(x, weight):
  B_, L_ = x.shape
  V, D = weight.shape
  N = B_ * L_
  info = plsc.get_sparse_core_info()
  NC, NS = info.num_cores, info.num_subcores
  NW = NC * NS
  per_w = N // NW
  n_ch = per_w // CH
  w_p = jnp.pad(weight, ((0, 0), (0, D)))
  idx = x.reshape(NW, n_ch, CH).astype(jnp.int32)
  out = _build(N, V, D, n_ch, NC, NS)(idx, w_p)
  return out.reshape(B_, L_, D)


# speedup vs baseline: 1.0048x; 1.0048x over previous
import functools

import jax
import jax.numpy as jnp
from jax import lax
from jax.experimental import pallas as pl
from jax.experimental.pallas import tpu as pltpu
from jax.experimental.pallas import tpu_sc as plsc

CH = 128
NBUF = 3


@functools.cache
def _build(N, V, D, n_ch, NC, NS):
  NW = NC * NS
  D2 = 2 * D
  mesh = plsc.VectorSubcoreMesh(core_axis_name="c", subcore_axis_name="s")

  @functools.partial(
      pl.kernel,
      mesh=mesh,
      out_type=jax.ShapeDtypeStruct((N // CH, CH // 2, D2), jnp.float32),
      scratch_types=[
          pltpu.VMEM((n_ch, CH), jnp.int32),
          pltpu.VMEM((NBUF, CH, D2), jnp.float32),
          pltpu.VMEM((NBUF, CH // 2, D2), jnp.float32),
          [pltpu.SemaphoreType.DMA] * NBUF,
          [pltpu.SemaphoreType.DMA] * NBUF,
      ],
  )
  def k(idx_hbm, table_hbm, out_hbm, idx_v, wide, comp, gsems, wsems):
    wid = lax.axis_index("s") * NC + lax.axis_index("c")
    base = wid * n_ch
    pltpu.sync_copy(idx_hbm.at[wid], idx_v)

    def gather(c, b):
      pltpu.async_copy(table_hbm.at[idx_v.at[c]], wide.at[b], gsems[b])

    def wait_gather(c, b):
      pltpu.make_async_copy(
          table_hbm.at[idx_v.at[c]], wide.at[b], gsems[b]).wait()

    def select(b):
      @pl.loop(0, CH // 16)
      def _(g):
        for rr in range(16):
          r = g * 16 + rr
          for j in range(D // 16):
            comp[b, r // 2, pl.ds((r % 2) * D + 16 * j, 16)] = (
                wide[b, r, pl.ds(16 * j, 16)])

    def write(c, b):
      pltpu.async_copy(comp.at[b], out_hbm.at[base + c], wsems[b])

    def wait_write(c, b):
      pltpu.make_async_copy(
          comp.at[b], out_hbm.at[base + c], wsems[b]).wait()

    gather(0, 0)
    gather(1, 1)

    n_loop = ((n_ch + NBUF - 1) // NBUF) * NBUF

    @pl.loop(0, n_loop, step=NBUF)
    def _(j):
      for b in range(NBUF):
        c = j + b

        @pl.when(jnp.logical_and(c - 2 >= 0, c - 2 < n_ch))
        def _():
          wait_write(c - 2, (b - 2) % NBUF)

        @pl.when(c + 2 < n_ch)
        def _():
          gather(c + 2, (b + 2) % NBUF)

        @pl.when(c < n_ch)
        def _():
          wait_gather(c, b)
          select(b)
          write(c, b)

    for c in range(max(0, n_loop - 2), n_ch):
      wait_write(c, c % NBUF)

  return k


def kernel(x, weight):
  B_, L_ = x.shape
  V, D = weight.shape
  N = B_ * L_
  info = plsc.get_sparse_core_info()
  NC, NS = info.num_cores, info.num_subcores
  NW = NC * NS
  per_w = N // NW
  n_ch = per_w // CH
  w_p = jnp.pad(weight, ((0, 0), (0, D)))
  idx = x.reshape(NW, n_ch, CH).astype(jnp.int32)
  out = _build(N, V, D, n_ch, NC, NS)(idx, w_p)
  return out.reshape(B_, L_, D)
